# baseline (speedup 1.0000x reference)
import functools

import jax
import jax.numpy as jnp
from jax import lax
from jax.experimental import pallas as pl
from jax.experimental.pallas import tpu as pltpu
from jax.experimental.pallas import tpu_sc as plsc

T = 2048
H = 1024
HP = H // 2
E = 8
K = 2
F = 2048
BT = 256
NP = 6144
NB = NP // BT
NBT = 32
CSB = 512


def _pack_bf16(v32):
    vb = v32.astype(jnp.bfloat16)
    n = vb.shape[1] // 2
    lo = lax.bitcast_convert_type(vb[:, :n], jnp.uint16).astype(jnp.uint32)
    hi = lax.bitcast_convert_type(vb[:, n:], jnp.uint16).astype(jnp.uint32)
    return lax.bitcast_convert_type(lo | (hi << 16), jnp.int32)


def _unpack_bf16(vi):
    u = lax.bitcast_convert_type(vi, jnp.uint32)
    lo = lax.bitcast_convert_type((u & 0xFFFF).astype(jnp.uint16),
                                  jnp.bfloat16)
    hi = lax.bitcast_convert_type((u >> 16).astype(jnp.uint16), jnp.bfloat16)
    return jnp.concatenate([lo, hi], axis=1)


RB = 512


def _router_body(x_ref, wr_ref, pos_ref, w1_ref, w2_ref, tbl_ref, xp_ref,
                 probs_s):
    i = pl.program_id(0)
    xb = x_ref[0]
    xp_ref[...] = _pack_bf16(xb)
    logits = jnp.dot(xb, wr_ref[...], preferred_element_type=jnp.float32)
    m = jnp.max(logits, axis=1, keepdims=True)
    ex = jnp.exp(logits - m)
    probs_s[pl.ds(i * RB, RB), :] = ex / jnp.sum(ex, axis=1, keepdims=True)

    @pl.when(i == T // RB - 1)
    def _():
        _router_tail(probs_s[...], pos_ref, w1_ref, w2_ref, tbl_ref)


def _router_tail(probs, pos_ref, w1_ref, w2_ref, tbl_ref):
    ii = lax.broadcasted_iota(jnp.int32, (T, E), 1)
    m1 = jnp.max(probs, axis=1, keepdims=True)
    i1 = jnp.min(jnp.where(probs == m1, ii, E), axis=1, keepdims=True)
    pmask = jnp.where(ii == i1, -1.0, probs)
    m2 = jnp.max(pmask, axis=1, keepdims=True)
    i2 = jnp.min(jnp.where(pmask == m2, ii, E), axis=1, keepdims=True)
    s = m1 + m2
    w1_ref[...] = m1 / s
    w2_ref[...] = m2 / s

    oh1 = (ii == i1).astype(jnp.float32)
    oh2 = (ii == i2).astype(jnp.float32)
    oh = jnp.concatenate([oh1, oh2], axis=0)

    r = lax.broadcasted_iota(jnp.int32, (CSB, CSB), 0)
    c = lax.broadcasted_iota(jnp.int32, (CSB, CSB), 1)
    L = (r >= c).astype(jnp.float32)
    carry = jnp.zeros((1, E), jnp.float32)
    segs = []
    for b in range(2 * T // CSB):
        seg = lax.slice(oh, (b * CSB, 0), ((b + 1) * CSB, E))
        cs = jnp.dot(L, seg, preferred_element_type=jnp.float32) + carry
        segs.append(cs)
        carry = lax.slice(cs, (CSB - 1, 0), (CSB, E))
    cums = jnp.concatenate(segs, axis=0)

    counts = carry
    pci = ((counts.astype(jnp.int32) + BT - 1) // BT) * BT

    ends = pci.astype(jnp.float32)
    for sh in (1, 2, 4):
        shifted = jnp.concatenate(
            [jnp.zeros((1, sh), jnp.float32), ends[:, :E - sh]], axis=1)
        ends = ends + shifted
    total = lax.slice(ends, (0, E - 1), (1, E))

    rowb = lax.broadcasted_iota(jnp.int32, (NBT, 1), 0)
    nbb = jnp.broadcast_to((total / BT).astype(jnp.int32), (NBT, 1))
    act = (rowb < nbb).astype(jnp.int32)
    xblk = jnp.where(rowb < nbb, rowb, nbb - 1)
    endsb = jnp.broadcast_to(ends, (NBT, E))
    wexp = jnp.sum(((xblk * BT).astype(jnp.float32) >= endsb)
                   .astype(jnp.int32), axis=1, keepdims=True)

    wef = wexp.astype(jnp.float32)
    prev = jnp.concatenate([jnp.full((1, 1), -1.0), wef[:NBT - 1]], axis=0)
    sw = (wef != prev).astype(jnp.float32)
    r = sw
    for sh in (1, 2, 4, 8, 16):
        r = r + jnp.concatenate(
            [jnp.zeros((sh, 1), jnp.float32), r[:NBT - sh]], axis=0)
    r = r - 1.0
    nruns = jnp.max(r, axis=0, keepdims=True) + 1.0
    slot = r.astype(jnp.int32) % 2
    ii0 = lax.broadcasted_iota(jnp.int32, (NBT, NBT), 0)
    ii1 = lax.broadcasted_iota(jnp.int32, (NBT, NBT), 1)
    eye = (ii0 == ii1).astype(jnp.float32)
    ones = jnp.ones((NBT, NBT), jnp.float32)
    r_cols = jnp.dot(ones, eye * r, preferred_element_type=jnp.float32)
    sw_cols = jnp.dot(ones, eye * sw, preferred_element_type=jnp.float32)
    we_cols = jnp.dot(ones, eye * wef, preferred_element_type=jnp.float32)
    mnext = ((sw_cols == 1.0) & (r_cols == r + 1.0)).astype(jnp.float32)
    pfe = jnp.sum(mnext * we_cols, axis=1, keepdims=True).astype(jnp.int32)
    pf = (sw * (r + 1.0 < nruns).astype(jnp.float32)).astype(jnp.int32)

    lane = lax.broadcasted_iota(jnp.int32, (NBT, E), 1)
    tbl_ref[...] = (xblk * (lane == 0).astype(jnp.int32)
                    + wexp * (lane == 1).astype(jnp.int32)
                    + act * (lane == 2).astype(jnp.int32)
                    + sw.astype(jnp.int32) * (lane == 3).astype(jnp.int32)
                    + slot * (lane == 4).astype(jnp.int32)
                    + pfe * (lane == 5).astype(jnp.int32)
                    + pf * (lane == 6).astype(jnp.int32))

    e_all = jnp.concatenate([i1, i2], axis=0)
    iia = lax.broadcasted_iota(jnp.int32, (2 * T, E), 1)
    pcf = pci.astype(jnp.float32)
    base = jnp.sum(jnp.where(iia < e_all, pcf, 0.0), axis=1, keepdims=True)
    rank = jnp.sum(oh * cums, axis=1, keepdims=True) - 1.0
    posv = rank + base

    rows = 2 * T // 128
    posi = posv.astype(jnp.int32)
    rowi = lax.broadcasted_iota(jnp.int32, (2 * T, 128), 0)
    lanei = lax.broadcasted_iota(jnp.int32, (2 * T, 128), 1)
    onrow = rowi % 128 == lanei
    p_hi = jnp.where(onrow, (posi // 256).astype(jnp.float32), 0.0)
    p_lo = jnp.where(onrow, (posi % 256).astype(jnp.float32), 0.0)
    sel0 = lax.broadcasted_iota(jnp.int32, (rows, 2 * T), 0)
    sel1 = lax.broadcasted_iota(jnp.int32, (rows, 2 * T), 1)
    s_mat = (sel1 // 128 == sel0).astype(jnp.float32)
    r_hi = jnp.dot(s_mat, p_hi, preferred_element_type=jnp.float32)
    r_lo = jnp.dot(s_mat, p_lo, preferred_element_type=jnp.float32)
    pos_ref[...] = r_hi.astype(jnp.int32) * 256 + r_lo.astype(jnp.int32)


def _run_router(hs3d, w_router):
    n = T // RB
    return pl.pallas_call(
        _router_body,
        grid=(n,),
        in_specs=[
            pl.BlockSpec((1, RB, H), lambda i: (0, i, 0)),
            pl.BlockSpec((H, E), lambda i: (0, 0)),
        ],
        out_specs=(
            pl.BlockSpec((2 * T // 128, 128), lambda i: (0, 0)),
            pl.BlockSpec((T, 1), lambda i: (0, 0)),
            pl.BlockSpec((T, 1), lambda i: (0, 0)),
            pl.BlockSpec((NBT, E), lambda i: (0, 0)),
            pl.BlockSpec((RB, HP), lambda i: (i, 0)),
        ),
        out_shape=(
            jax.ShapeDtypeStruct((2 * T // 128, 128), jnp.int32),
            jax.ShapeDtypeStruct((T, 1), jnp.float32),
            jax.ShapeDtypeStruct((T, 1), jnp.float32),
            jax.ShapeDtypeStruct((NBT, E), jnp.int32),
            jax.ShapeDtypeStruct((T, HP), jnp.int32),
        ),
        scratch_shapes=[pltpu.VMEM((T, E), jnp.float32)],
    )(hs3d, w_router)


def _make_sc_mesh():
    return plsc.VectorSubcoreMesh(core_axis_name="c", subcore_axis_name="s")


def _dispatch_sc(xp, pos):
    info = plsc.get_sparse_core_info()
    nw = info.num_cores * info.num_subcores
    per_w = 2 * T // nw
    cn = 64

    @functools.partial(
        pl.kernel,
        out_type=jax.ShapeDtypeStruct((NP, HP), jnp.int32),
        mesh=_make_sc_mesh(),
        scratch_types=[
            pltpu.VMEM((cn,), jnp.int32),
            pltpu.VMEM((cn, HP), jnp.int32),
            pltpu.SemaphoreType.DMA,
        ],
    )
    def k(xp_hbm, pos_hbm, xg_hbm, idx_v, buf_v, sem):
        wid = lax.axis_index("s") * info.num_cores + lax.axis_index("c")
        for cc in range(per_w // cn):
            i0 = wid * per_w + cc * cn
            base = lax.rem(i0, T)
            pltpu.sync_copy(pos_hbm.at[pl.ds(i0, cn)], idx_v)
            pltpu.sync_copy(xp_hbm.at[pl.ds(base, cn)], buf_v)
            pltpu.async_copy(buf_v, xg_hbm.at[idx_v], sem).wait()

    return k(xp, pos)


def _combine_gather_sc(y, pos):
    info = plsc.get_sparse_core_info()
    nw = info.num_cores * info.num_subcores
    per_w = T // nw
    cn = 64

    @functools.partial(
        pl.kernel,
        out_type=(
            jax.ShapeDtypeStruct((T, HP), jnp.int32),
            jax.ShapeDtypeStruct((T, HP), jnp.int32),
        ),
        mesh=_make_sc_mesh(),
        scratch_types=[
            pltpu.VMEM((cn,), jnp.int32),
            pltpu.VMEM((cn, HP), jnp.int32),
            pltpu.SemaphoreType.DMA,
        ],
    )
    def k(y_hbm, pos_hbm, y1_hbm, y2_hbm, idx_v, buf_v, sem):
        wid = lax.axis_index("s") * info.num_cores + lax.axis_index("c")
        for half, out_hbm in ((0, y1_hbm), (1, y2_hbm)):
            for cc in range(per_w // cn):
                t0 = wid * per_w + cc * cn
                pltpu.sync_copy(pos_hbm.at[pl.ds(half * T + t0, cn)], idx_v)
                pltpu.async_copy(y_hbm.at[idx_v], buf_v, sem).wait()
                pltpu.sync_copy(buf_v, out_hbm.at[pl.ds(t0, cn)])

    return k(y, pos)


def _ffn_body(tbl_ref, xg_ref, wg_hbm, wu_hbm, wd_hbm, y_ref,
              wgb, wub, wdb, sems):
    b = pl.program_id(0)
    act = tbl_ref[b, 2]
    sw = tbl_ref[b, 3]
    slot = tbl_ref[b, 4]

    @pl.when(b == 0)
    def _():
        e0 = tbl_ref[0, 1]
        pltpu.make_async_copy(wg_hbm.at[e0], wgb.at[0], sems.at[0, 0]).start()
        pltpu.make_async_copy(wu_hbm.at[e0], wub.at[0], sems.at[1, 0]).start()
        pltpu.make_async_copy(wd_hbm.at[e0], wdb.at[0], sems.at[2, 0]).start()

    @pl.when((tbl_ref[b, 6] == 1) & (act == 1))
    def _():
        ns = 1 - slot
        e = tbl_ref[b, 5]
        pltpu.make_async_copy(wg_hbm.at[e], wgb.at[ns], sems.at[0, ns]).start()
        pltpu.make_async_copy(wu_hbm.at[e], wub.at[ns], sems.at[1, ns]).start()
        pltpu.make_async_copy(wd_hbm.at[e], wdb.at[ns], sems.at[2, ns]).start()

    @pl.when((sw == 1) & (act == 1))
    def _():
        pltpu.make_async_copy(wg_hbm.at[0], wgb.at[slot],
                              sems.at[0, slot]).wait()
        pltpu.make_async_copy(wu_hbm.at[0], wub.at[slot],
                              sems.at[1, slot]).wait()
        pltpu.make_async_copy(wd_hbm.at[0], wdb.at[slot],
                              sems.at[2, slot]).wait()

    @pl.when(act == 1)
    def _():
        x = _unpack_bf16(xg_ref[...]).astype(jnp.float32)
        g = jnp.dot(x, wgb[slot], preferred_element_type=jnp.float32)
        u = jnp.dot(x, wub[slot], preferred_element_type=jnp.float32)
        a = (g / (1.0 + jnp.exp(-g))) * u
        y = jnp.dot(a, wdb[slot], preferred_element_type=jnp.float32)
        y_ref[...] = _pack_bf16(y)


def _run_ffn(xg, w_gate, w_up, w_down, tbl):
    hbm = pl.BlockSpec(memory_space=pltpu.MemorySpace.HBM)
    grid_spec = pltpu.PrefetchScalarGridSpec(
        num_scalar_prefetch=1,
        grid=(NB,),
        in_specs=[
            pl.BlockSpec((BT, HP), lambda b, tbl: (tbl[b, 0], 0)),
            hbm, hbm, hbm,
        ],
        out_specs=pl.BlockSpec((BT, HP), lambda b, tbl: (tbl[b, 0], 0)),
        scratch_shapes=[
            pltpu.VMEM((2, H, F), jnp.float32),
            pltpu.VMEM((2, H, F), jnp.float32),
            pltpu.VMEM((2, F, H), jnp.float32),
            pltpu.SemaphoreType.DMA((3, 2)),
        ],
    )
    return pl.pallas_call(
        _ffn_body,
        grid_spec=grid_spec,
        out_shape=jax.ShapeDtypeStruct((NP, HP), jnp.int32),
    )(tbl, xg, w_gate, w_up, w_down)


def _combine_body(w1_ref, w2_ref, y1_ref, y2_ref, out_ref):
    y1 = _unpack_bf16(y1_ref[...]).astype(jnp.float32)
    y2 = _unpack_bf16(y2_ref[...]).astype(jnp.float32)
    out_ref[...] = w1_ref[...] * y1 + w2_ref[...] * y2


def _run_combine(w1, w2, y1, y2):
    return pl.pallas_call(
        _combine_body,
        grid=(T // BT,),
        in_specs=[
            pl.BlockSpec((BT, 1), lambda i: (i, 0)),
            pl.BlockSpec((BT, 1), lambda i: (i, 0)),
            pl.BlockSpec((BT, HP), lambda i: (i, 0)),
            pl.BlockSpec((BT, HP), lambda i: (i, 0)),
        ],
        out_specs=pl.BlockSpec((BT, H), lambda i: (i, 0)),
        out_shape=jax.ShapeDtypeStruct((T, H), jnp.float32),
    )(w1, w2, y1, y2)


def kernel(hidden_states, W_router, W_gate, W_up, W_down):
    b, s, h = hidden_states.shape

    pos2d, w1, w2, tbl, xp = _run_router(hidden_states, W_router)
    pos = pos2d.reshape(2 * T)

    xg = _dispatch_sc(xp, pos)
    y = _run_ffn(xg, W_gate, W_up, W_down, tbl)
    th = T // 2
    y1, y2 = _combine_gather_sc(y, pos)
    out = _run_combine(w1, w2, y1, y2)
    return out.reshape(b, s, h)


# speedup vs baseline: 1.0112x; 1.0112x over previous
import functools

import jax
import jax.numpy as jnp
from jax import lax
from jax.experimental import pallas as pl
from jax.experimental.pallas import tpu as pltpu
from jax.experimental.pallas import tpu_sc as plsc

T = 2048
H = 1024
HP = H // 2
E = 8
K = 2
F = 2048
BT = 256
NP = 6144
NB = NP // BT
NBT = 32
CSB = 512


def _pack_bf16(v32):
    vb = v32.astype(jnp.bfloat16)
    n = vb.shape[1] // 2
    lo = lax.bitcast_convert_type(vb[:, :n], jnp.uint16).astype(jnp.uint32)
    hi = lax.bitcast_convert_type(vb[:, n:], jnp.uint16).astype(jnp.uint32)
    return lax.bitcast_convert_type(lo | (hi << 16), jnp.int32)


def _unpack_bf16(vi):
    u = lax.bitcast_convert_type(vi, jnp.uint32)
    lo = lax.bitcast_convert_type((u & 0xFFFF).astype(jnp.uint16),
                                  jnp.bfloat16)
    hi = lax.bitcast_convert_type((u >> 16).astype(jnp.uint16), jnp.bfloat16)
    return jnp.concatenate([lo, hi], axis=1)


RB = 512


def _router_body(x_ref, wr_ref, pos_ref, w1_ref, w2_ref, tbl_ref, xp_ref,
                 probs_s):
    i = pl.program_id(0)
    xb = x_ref[0]
    xp_ref[...] = _pack_bf16(xb)
    logits = jnp.dot(xb, wr_ref[...], preferred_element_type=jnp.float32)
    m = jnp.max(logits, axis=1, keepdims=True)
    ex = jnp.exp(logits - m)
    probs_s[pl.ds(i * RB, RB), :] = ex / jnp.sum(ex, axis=1, keepdims=True)

    @pl.when(i == T // RB - 1)
    def _():
        _router_tail(probs_s[...], pos_ref, w1_ref, w2_ref, tbl_ref)


def _router_tail(probs, pos_ref, w1_ref, w2_ref, tbl_ref):
    ii = lax.broadcasted_iota(jnp.int32, (T, E), 1)
    m1 = jnp.max(probs, axis=1, keepdims=True)
    i1 = jnp.min(jnp.where(probs == m1, ii, E), axis=1, keepdims=True)
    pmask = jnp.where(ii == i1, -1.0, probs)
    m2 = jnp.max(pmask, axis=1, keepdims=True)
    i2 = jnp.min(jnp.where(pmask == m2, ii, E), axis=1, keepdims=True)
    s = m1 + m2
    w1_ref[...] = m1 / s
    w2_ref[...] = m2 / s

    oh1 = (ii == i1).astype(jnp.float32)
    oh2 = (ii == i2).astype(jnp.float32)
    oh = jnp.concatenate([oh1, oh2], axis=0)

    r = lax.broadcasted_iota(jnp.int32, (CSB, CSB), 0)
    c = lax.broadcasted_iota(jnp.int32, (CSB, CSB), 1)
    L = (r >= c).astype(jnp.float32)
    carry = jnp.zeros((1, E), jnp.float32)
    segs = []
    for b in range(2 * T // CSB):
        seg = lax.slice(oh, (b * CSB, 0), ((b + 1) * CSB, E))
        cs = jnp.dot(L, seg, preferred_element_type=jnp.float32) + carry
        segs.append(cs)
        carry = lax.slice(cs, (CSB - 1, 0), (CSB, E))
    cums = jnp.concatenate(segs, axis=0)

    counts = carry
    pci = ((counts.astype(jnp.int32) + BT - 1) // BT) * BT

    ends = pci.astype(jnp.float32)
    for sh in (1, 2, 4):
        shifted = jnp.concatenate(
            [jnp.zeros((1, sh), jnp.float32), ends[:, :E - sh]], axis=1)
        ends = ends + shifted
    total = lax.slice(ends, (0, E - 1), (1, E))

    rowb = lax.broadcasted_iota(jnp.int32, (NBT, 1), 0)
    nbb = jnp.broadcast_to((total / BT).astype(jnp.int32), (NBT, 1))
    act = (rowb < nbb).astype(jnp.int32)
    xblk = jnp.where(rowb < nbb, rowb, nbb - 1)
    endsb = jnp.broadcast_to(ends, (NBT, E))
    wexp = jnp.sum(((xblk * BT).astype(jnp.float32) >= endsb)
                   .astype(jnp.int32), axis=1, keepdims=True)

    wef = wexp.astype(jnp.float32)
    prev = jnp.concatenate([jnp.full((1, 1), -1.0), wef[:NBT - 1]], axis=0)
    sw = (wef != prev).astype(jnp.float32)
    r = sw
    for sh in (1, 2, 4, 8, 16):
        r = r + jnp.concatenate(
            [jnp.zeros((sh, 1), jnp.float32), r[:NBT - sh]], axis=0)
    r = r - 1.0
    nruns = jnp.max(r, axis=0, keepdims=True) + 1.0
    slot = r.astype(jnp.int32) % 2
    ii0 = lax.broadcasted_iota(jnp.int32, (NBT, NBT), 0)
    ii1 = lax.broadcasted_iota(jnp.int32, (NBT, NBT), 1)
    eye = (ii0 == ii1).astype(jnp.float32)
    ones = jnp.ones((NBT, NBT), jnp.float32)
    r_cols = jnp.dot(ones, eye * r, preferred_element_type=jnp.float32)
    sw_cols = jnp.dot(ones, eye * sw, preferred_element_type=jnp.float32)
    we_cols = jnp.dot(ones, eye * wef, preferred_element_type=jnp.float32)
    mnext = ((sw_cols == 1.0) & (r_cols == r + 1.0)).astype(jnp.float32)
    pfe = jnp.sum(mnext * we_cols, axis=1, keepdims=True).astype(jnp.int32)
    pf = (sw * (r + 1.0 < nruns).astype(jnp.float32)).astype(jnp.int32)

    lane = lax.broadcasted_iota(jnp.int32, (NBT, E), 1)
    tbl_ref[...] = (xblk * (lane == 0).astype(jnp.int32)
                    + wexp * (lane == 1).astype(jnp.int32)
                    + act * (lane == 2).astype(jnp.int32)
                    + sw.astype(jnp.int32) * (lane == 3).astype(jnp.int32)
                    + slot * (lane == 4).astype(jnp.int32)
                    + pfe * (lane == 5).astype(jnp.int32)
                    + pf * (lane == 6).astype(jnp.int32))

    e_all = jnp.concatenate([i1, i2], axis=0)
    iia = lax.broadcasted_iota(jnp.int32, (2 * T, E), 1)
    pcf = pci.astype(jnp.float32)
    base = jnp.sum(jnp.where(iia < e_all, pcf, 0.0), axis=1, keepdims=True)
    rank = jnp.sum(oh * cums, axis=1, keepdims=True) - 1.0
    pos_ref[...] = (rank + base).astype(jnp.int32)


def _run_router(hs3d, w_router):
    n = T // RB
    return pl.pallas_call(
        _router_body,
        grid=(n,),
        in_specs=[
            pl.BlockSpec((1, RB, H), lambda i: (0, i, 0)),
            pl.BlockSpec((H, E), lambda i: (0, 0)),
        ],
        out_specs=(
            pl.BlockSpec((2 * T, 1), lambda i: (0, 0)),
            pl.BlockSpec((T, 1), lambda i: (0, 0)),
            pl.BlockSpec((T, 1), lambda i: (0, 0)),
            pl.BlockSpec((NBT, E), lambda i: (0, 0)),
            pl.BlockSpec((RB, HP), lambda i: (i, 0)),
        ),
        out_shape=(
            jax.ShapeDtypeStruct((2 * T, 1), jnp.int32),
            jax.ShapeDtypeStruct((T, 1), jnp.float32),
            jax.ShapeDtypeStruct((T, 1), jnp.float32),
            jax.ShapeDtypeStruct((NBT, E), jnp.int32),
            jax.ShapeDtypeStruct((T, HP), jnp.int32),
        ),
        scratch_shapes=[pltpu.VMEM((T, E), jnp.float32)],
    )(hs3d, w_router)


def _make_sc_mesh():
    return plsc.VectorSubcoreMesh(core_axis_name="c", subcore_axis_name="s")


def _dispatch_sc(xp, pos):
    info = plsc.get_sparse_core_info()
    nw = info.num_cores * info.num_subcores
    per_w = 2 * T // nw
    cn = 64

    @functools.partial(
        pl.kernel,
        out_type=jax.ShapeDtypeStruct((NP, HP), jnp.int32),
        mesh=_make_sc_mesh(),
        scratch_types=[
            pltpu.VMEM((cn,), jnp.int32),
            pltpu.VMEM((cn, HP), jnp.int32),
            pltpu.SemaphoreType.DMA,
        ],
    )
    def k(xp_hbm, pos_hbm, xg_hbm, idx_v, buf_v, sem):
        wid = lax.axis_index("s") * info.num_cores + lax.axis_index("c")
        for cc in range(per_w // cn):
            i0 = wid * per_w + cc * cn
            base = lax.rem(i0, T)
            pltpu.sync_copy(pos_hbm.at[pl.ds(i0, cn)], idx_v)
            pltpu.sync_copy(xp_hbm.at[pl.ds(base, cn)], buf_v)
            pltpu.async_copy(buf_v, xg_hbm.at[idx_v], sem).wait()

    return k(xp, pos)


def _combine_gather_sc(y, pos):
    info = plsc.get_sparse_core_info()
    nw = info.num_cores * info.num_subcores
    per_w = T // nw
    cn = 64

    @functools.partial(
        pl.kernel,
        out_type=(
            jax.ShapeDtypeStruct((T, HP), jnp.int32),
            jax.ShapeDtypeStruct((T, HP), jnp.int32),
        ),
        mesh=_make_sc_mesh(),
        scratch_types=[
            pltpu.VMEM((cn,), jnp.int32),
            pltpu.VMEM((cn, HP), jnp.int32),
            pltpu.SemaphoreType.DMA,
        ],
    )
    def k(y_hbm, pos_hbm, y1_hbm, y2_hbm, idx_v, buf_v, sem):
        wid = lax.axis_index("s") * info.num_cores + lax.axis_index("c")
        for half, out_hbm in ((0, y1_hbm), (1, y2_hbm)):
            for cc in range(per_w // cn):
                t0 = wid * per_w + cc * cn
                pltpu.sync_copy(pos_hbm.at[pl.ds(half * T + t0, cn)], idx_v)
                pltpu.async_copy(y_hbm.at[idx_v], buf_v, sem).wait()
                pltpu.sync_copy(buf_v, out_hbm.at[pl.ds(t0, cn)])

    return k(y, pos)


def _ffn_body(tbl_ref, xg_ref, wg_hbm, wu_hbm, wd_hbm, y_ref,
              wgb, wub, wdb, sems):
    b = pl.program_id(0)
    act = tbl_ref[b, 2]
    sw = tbl_ref[b, 3]
    slot = tbl_ref[b, 4]

    @pl.when(b == 0)
    def _():
        e0 = tbl_ref[0, 1]
        pltpu.make_async_copy(wg_hbm.at[e0], wgb.at[0], sems.at[0, 0]).start()
        pltpu.make_async_copy(wu_hbm.at[e0], wub.at[0], sems.at[1, 0]).start()
        pltpu.make_async_copy(wd_hbm.at[e0], wdb.at[0], sems.at[2, 0]).start()

    @pl.when((tbl_ref[b, 6] == 1) & (act == 1))
    def _():
        ns = 1 - slot
        e = tbl_ref[b, 5]
        pltpu.make_async_copy(wg_hbm.at[e], wgb.at[ns], sems.at[0, ns]).start()
        pltpu.make_async_copy(wu_hbm.at[e], wub.at[ns], sems.at[1, ns]).start()
        pltpu.make_async_copy(wd_hbm.at[e], wdb.at[ns], sems.at[2, ns]).start()

    @pl.when((sw == 1) & (act == 1))
    def _():
        pltpu.make_async_copy(wg_hbm.at[0], wgb.at[slot],
                              sems.at[0, slot]).wait()
        pltpu.make_async_copy(wu_hbm.at[0], wub.at[slot],
                              sems.at[1, slot]).wait()
        pltpu.make_async_copy(wd_hbm.at[0], wdb.at[slot],
                              sems.at[2, slot]).wait()

    @pl.when(act == 1)
    def _():
        x = _unpack_bf16(xg_ref[...]).astype(jnp.float32)
        g = jnp.dot(x, wgb[slot], preferred_element_type=jnp.float32)
        u = jnp.dot(x, wub[slot], preferred_element_type=jnp.float32)
        a = (g / (1.0 + jnp.exp(-g))) * u
        y = jnp.dot(a, wdb[slot], preferred_element_type=jnp.float32)
        y_ref[...] = _pack_bf16(y)


def _run_ffn(xg, w_gate, w_up, w_down, tbl):
    hbm = pl.BlockSpec(memory_space=pltpu.MemorySpace.HBM)
    grid_spec = pltpu.PrefetchScalarGridSpec(
        num_scalar_prefetch=1,
        grid=(NB,),
        in_specs=[
            pl.BlockSpec((BT, HP), lambda b, tbl: (tbl[b, 0], 0)),
            hbm, hbm, hbm,
        ],
        out_specs=pl.BlockSpec((BT, HP), lambda b, tbl: (tbl[b, 0], 0)),
        scratch_shapes=[
            pltpu.VMEM((2, H, F), jnp.float32),
            pltpu.VMEM((2, H, F), jnp.float32),
            pltpu.VMEM((2, F, H), jnp.float32),
            pltpu.SemaphoreType.DMA((3, 2)),
        ],
    )
    return pl.pallas_call(
        _ffn_body,
        grid_spec=grid_spec,
        out_shape=jax.ShapeDtypeStruct((NP, HP), jnp.int32),
    )(tbl, xg, w_gate, w_up, w_down)


def _combine_body(w1_ref, w2_ref, y1_ref, y2_ref, out_ref):
    y1 = _unpack_bf16(y1_ref[...]).astype(jnp.float32)
    y2 = _unpack_bf16(y2_ref[...]).astype(jnp.float32)
    out_ref[...] = w1_ref[...] * y1 + w2_ref[...] * y2


def _run_combine(w1, w2, y1, y2):
    return pl.pallas_call(
        _combine_body,
        grid=(T // BT,),
        in_specs=[
            pl.BlockSpec((BT, 1), lambda i: (i, 0)),
            pl.BlockSpec((BT, 1), lambda i: (i, 0)),
            pl.BlockSpec((BT, HP), lambda i: (i, 0)),
            pl.BlockSpec((BT, HP), lambda i: (i, 0)),
        ],
        out_specs=pl.BlockSpec((BT, H), lambda i: (i, 0)),
        out_shape=jax.ShapeDtypeStruct((T, H), jnp.float32),
    )(w1, w2, y1, y2)


def kernel(hidden_states, W_router, W_gate, W_up, W_down):
    b, s, h = hidden_states.shape

    pos2d, w1, w2, tbl, xp = _run_router(hidden_states, W_router)
    pos = pos2d.reshape(2 * T)

    xg = _dispatch_sc(xp, pos)
    y = _run_ffn(xg, W_gate, W_up, W_down, tbl)
    th = T // 2
    y1, y2 = _combine_gather_sc(y, pos)
    out = _run_combine(w1, w2, y1, y2)
    return out.reshape(b, s, h)
